# Initial kernel scaffold; baseline (speedup 1.0000x reference)
import jax
import jax.numpy as jnp
from jax.experimental import pallas as pl


def kernel(terrains, hut_colors, hut_rotations, windmill_rotations, tower_rotations, tent_rotations, tree_types, plant_types, prop_types, W):
    raise NotImplementedError("write your pallas kernel here")


# speedup vs baseline: 42.3077x; 42.3077x over previous
import functools

import jax
import jax.numpy as jnp
from jax import lax
from jax.experimental import pallas as pl
from jax.experimental.pallas import tpu as pltpu
from jax.experimental.pallas import tpu_sc as plsc

_B = 1024
_HW = 625
_E = 16
_NW = 32
_BPW = _B // _NW
_CHUNK = 8
_NCHUNK = _BPW // _CHUNK

_T1_ROWS = 6 * 5 * 4 * 4
_T2_ROWS = 4 * 4 * 3 * 3 * 8


def _sc_body(t1h, t2h, i0, i1, i2, i3, i4, i5, i6, i7, i8, outh,
             t1v, t2v, v0, v1, v2, v3, v4, v5, v6, v7, v8, outv):
    ihs = (i0, i1, i2, i3, i4, i5, i6, i7, i8)
    ivs = (v0, v1, v2, v3, v4, v5, v6, v7, v8)
    wid = lax.axis_index("s") * 2 + lax.axis_index("c")

    pltpu.sync_copy(t1h, t1v)
    pltpu.sync_copy(t2h, t2v)
    iota = lax.iota(jnp.int32, 16)

    def chunk_body(ck, carry):
        b0 = wid * _BPW + ck * _CHUNK
        coff = b0 * _HW
        for ih, iv in zip(ihs, ivs):
            pltpu.sync_copy(ih.at[pl.ds(coff, _CHUNK * _HW)], iv)

        def batch_body(bi, carry):
            base = bi * _HW

            def group_body(g, carry):
                go = jnp.minimum(g * 16, _HW - 16)
                cell = iota + (base + go)
                ts = [plsc.load_gather(iv, [cell]) for iv in ivs]
                q1 = ((ts[0] * 5 + ts[1]) * 4 + ts[2]) * 4 + ts[3]
                q2 = (((ts[4] * 4 + ts[5]) * 3 + ts[6]) * 3 + ts[7]) * 8 + ts[8]
                q1s = q1 * 16
                q2s = q2 * 16
                st0 = iota + go
                for e in range(_E):
                    r = (plsc.load_gather(t1v, [q1s + e]) +
                         plsc.load_gather(t2v, [q2s + e]))
                    plsc.store_scatter(outv, [st0 + e * _HW], r)
                return carry

            lax.fori_loop(0, (_HW + 15) // 16, group_body, 0)
            pltpu.sync_copy(outv, outh.at[b0 + bi])
            return carry

        lax.fori_loop(0, _CHUNK, batch_body, 0)
        return carry

    lax.fori_loop(0, _NCHUNK, chunk_body, 0)


@jax.jit
def _sc_call(t1, t2, *idx_flat):
    mesh = plsc.VectorSubcoreMesh(core_axis_name="c", subcore_axis_name="s")
    scratch = (
        [pltpu.VMEM((_T1_ROWS * _E,), jnp.float32),
         pltpu.VMEM((_T2_ROWS * _E,), jnp.float32)]
        + [pltpu.VMEM((_CHUNK * _HW,), jnp.int32) for _ in range(9)]
        + [pltpu.VMEM((_E * _HW,), jnp.float32)]
    )
    f = pl.kernel(
        _sc_body,
        out_type=jax.ShapeDtypeStruct((_B, _E * _HW), jnp.float32),
        mesh=mesh,
        scratch_types=scratch,
        compiler_params=pltpu.CompilerParams(needs_layout_passes=False),
    )
    return f(t1, t2, *idx_flat)


def kernel(terrains, hut_colors, hut_rotations, windmill_rotations,
           tower_rotations, tent_rotations, tree_types, plant_types,
           prop_types, W):
    T1 = (W[0:6][:, None, None, None, :]
          + W[6:11][None, :, None, None, :]
          + W[11:15][None, None, :, None, :]
          + W[15:19][None, None, None, :, :]).reshape(_T1_ROWS * _E)
    T2 = (W[19:23][:, None, None, None, None, :]
          + W[23:27][None, :, None, None, None, :]
          + W[27:30][None, None, :, None, None, :]
          + W[30:33][None, None, None, :, None, :]
          + W[33:41][None, None, None, None, :, :]).reshape(_T2_ROWS * _E)
    props = (terrains, hut_colors, hut_rotations, windmill_rotations,
             tower_rotations, tent_rotations, tree_types, plant_types,
             prop_types)
    idx_flat = [p.astype(jnp.int32).reshape(_B * _HW) for p in props]
    out = _sc_call(T1, T2, *idx_flat)
    return out.reshape(_B, _E, 25, 25)
